# baseline (speedup 1.0000x reference)
import functools

import jax
import jax.numpy as jnp
from jax import lax
from jax.experimental import pallas as pl
from jax.experimental.pallas import tpu as pltpu
from jax.experimental.pallas import tpu_sc as plsc

K = 64
B, S, D = 4, 2048, 4096
NCORES = 2
NSUB = 16
ROWS_PER_SUB = (B * S) // (NCORES * NSUB)
PF_PER_SUB = 16
SCATTER_WORKERS = K // PF_PER_SUB

_mesh = plsc.VectorSubcoreMesh(core_axis_name="c", subcore_axis_name="s")


@functools.partial(
    pl.kernel,
    out_type=jax.ShapeDtypeStruct((B * S, D), jnp.float32),
    mesh=_mesh,
    scratch_types=[
        pltpu.VMEM((16,), jnp.int32),
        pltpu.VMEM((16,), jnp.int32),
        pltpu.VMEM((PF_PER_SUB, D), jnp.float32),
        pltpu.SemaphoreType.DMA,
    ],
)
def _postfix_kernel(x_hbm, seq_hbm, pf_hbm, out_hbm, seq_v, idx_v, rows_v, sem):
    c = lax.axis_index("c")
    s = lax.axis_index("s")

    wid = c * NSUB + s
    row0 = wid * ROWS_PER_SUB
    pltpu.sync_copy(x_hbm.at[pl.ds(row0, ROWS_PER_SUB)],
                    out_hbm.at[pl.ds(row0, ROWS_PER_SUB)])

    plsc.subcore_barrier()

    sm = s % (NSUB // 2)
    @pl.when(sm < SCATTER_WORKERS)
    def _():
        b = 2 * c + s // (NSUB // 2)
        j = sm
        pltpu.sync_copy(seq_hbm.at[b], seq_v)
        pltpu.sync_copy(pf_hbm.at[pl.ds(j * PF_PER_SUB, PF_PER_SUB)], rows_v)
        lane = lax.iota(jnp.int32, 16)
        idx_v[...] = seq_v[...] + b * S + j * PF_PER_SUB + lane
        pltpu.async_copy(rows_v, out_hbm.at[idx_v], sem).wait()


def kernel(crossattn_emb, crossattn_seqlens, postfix_embeds):
    x2d = crossattn_emb.reshape(B * S, D)
    seq_bcast = jnp.broadcast_to(
        crossattn_seqlens.astype(jnp.int32)[:, None], (B, 16))
    out2d = _postfix_kernel(x2d, seq_bcast, postfix_embeds)
    return out2d.reshape(B, S, D)


# speedup vs baseline: 33.1322x; 33.1322x over previous
import functools

import jax
import jax.numpy as jnp
from jax import lax
from jax.experimental import pallas as pl
from jax.experimental.pallas import tpu as pltpu
from jax.experimental.pallas import tpu_sc as plsc

K = 64
B, S, D = 4, 2048, 4096
NCORES = 2
NSUB = 16
ROWS_PER_SUB = (B * S) // (NCORES * NSUB)
CH = 8
NCH = ROWS_PER_SUB // CH
PF_PER_SUB = 16
SCATTER_WORKERS = K // PF_PER_SUB

_mesh = plsc.VectorSubcoreMesh(core_axis_name="c", subcore_axis_name="s")


@functools.partial(
    pl.kernel,
    out_type=jax.ShapeDtypeStruct((B * S, D), jnp.float32),
    mesh=_mesh,
    scratch_types=[
        pltpu.VMEM((2 * CH, D), jnp.float32),
        pltpu.VMEM((16,), jnp.int32),
        pltpu.VMEM((16,), jnp.int32),
        pltpu.SemaphoreType.DMA,
        pltpu.SemaphoreType.DMA,
        pltpu.SemaphoreType.DMA,
        pltpu.SemaphoreType.DMA,
        pltpu.SemaphoreType.DMA,
    ],
)
def _postfix_kernel(x_hbm, seq_hbm, pf_hbm, out_hbm,
                    buf, seq_v, idx_v, g0, g1, s0, s1, psem):
    c = lax.axis_index("c")
    s = lax.axis_index("s")
    wid = c * NSUB + s
    row0 = wid * ROWS_PER_SUB
    buf_a = buf.at[pl.ds(0, CH)]
    buf_b = buf.at[pl.ds(CH, CH)]

    def rows(i):
        return pl.ds(row0 + i * CH, CH)

    pltpu.make_async_copy(x_hbm.at[rows(0)], buf_a, g0).start()
    pltpu.make_async_copy(x_hbm.at[rows(1)], buf_b, g1).start()

    @pl.loop(0, NCH // 2 - 1)
    def _pipe(i):
        c0 = 2 * i
        pltpu.make_async_copy(x_hbm.at[rows(c0)], buf_a, g0).wait()
        pltpu.make_async_copy(buf_a, out_hbm.at[rows(c0)], s0).start()
        pltpu.make_async_copy(x_hbm.at[rows(c0 + 1)], buf_b, g1).wait()
        pltpu.make_async_copy(buf_b, out_hbm.at[rows(c0 + 1)], s1).start()
        pltpu.make_async_copy(buf_a, out_hbm.at[rows(c0)], s0).wait()
        pltpu.make_async_copy(x_hbm.at[rows(c0 + 2)], buf_a, g0).start()
        pltpu.make_async_copy(buf_b, out_hbm.at[rows(c0 + 1)], s1).wait()
        pltpu.make_async_copy(x_hbm.at[rows(c0 + 3)], buf_b, g1).start()

    last = NCH - 2
    pltpu.make_async_copy(x_hbm.at[rows(last)], buf_a, g0).wait()
    pltpu.make_async_copy(buf_a, out_hbm.at[rows(last)], s0).start()
    pltpu.make_async_copy(x_hbm.at[rows(last + 1)], buf_b, g1).wait()
    pltpu.make_async_copy(buf_b, out_hbm.at[rows(last + 1)], s1).start()
    pltpu.make_async_copy(buf_a, out_hbm.at[rows(last)], s0).wait()
    pltpu.make_async_copy(buf_b, out_hbm.at[rows(last + 1)], s1).wait()

    plsc.subcore_barrier()

    sm = s % (NSUB // 2)
    @pl.when(sm < SCATTER_WORKERS)
    def _():
        b = 2 * c + s // (NSUB // 2)
        j = sm
        pltpu.sync_copy(seq_hbm.at[b], seq_v)
        pltpu.sync_copy(pf_hbm.at[pl.ds(j * PF_PER_SUB, PF_PER_SUB)], buf)
        lane = lax.iota(jnp.int32, 16)
        idx_v[...] = seq_v[...] + b * S + j * PF_PER_SUB + lane
        pltpu.async_copy(buf, out_hbm.at[idx_v], psem).wait()


def kernel(crossattn_emb, crossattn_seqlens, postfix_embeds):
    x2d = crossattn_emb.reshape(B * S, D)
    seq_bcast = jnp.broadcast_to(
        crossattn_seqlens.astype(jnp.int32)[:, None], (B, 16))
    out2d = _postfix_kernel(x2d, seq_bcast, postfix_embeds)
    return out2d.reshape(B, S, D)


# speedup vs baseline: 34.8445x; 1.0517x over previous
import functools

import jax
import jax.numpy as jnp
from jax import lax
from jax.experimental import pallas as pl
from jax.experimental.pallas import tpu as pltpu
from jax.experimental.pallas import tpu_sc as plsc

K = 64
B, S, D = 4, 2048, 4096
NCORES = 2
NSUB = 16
ROWS_PER_SUB = (B * S) // (NCORES * NSUB)
CH = 4
NCH = ROWS_PER_SUB // CH
NSLOT = 4
NBLK = NCH // NSLOT
PF_PER_SUB = 16
SCATTER_WORKERS = K // PF_PER_SUB

_mesh = plsc.VectorSubcoreMesh(core_axis_name="c", subcore_axis_name="s")


@functools.partial(
    pl.kernel,
    out_type=jax.ShapeDtypeStruct((B * S, D), jnp.float32),
    mesh=_mesh,
    scratch_types=[
        pltpu.VMEM((NSLOT * CH, D), jnp.float32),
        pltpu.VMEM((16,), jnp.int32),
        pltpu.VMEM((16,), jnp.int32),
        [pltpu.SemaphoreType.DMA] * NSLOT,
        [pltpu.SemaphoreType.DMA] * NSLOT,
        pltpu.SemaphoreType.DMA,
    ],
)
def _postfix_kernel(x_hbm, seq_hbm, pf_hbm, out_hbm,
                    buf, seq_v, idx_v, gsem, ssem, psem):
    c = lax.axis_index("c")
    s = lax.axis_index("s")
    wid = c * NSUB + s
    row0 = wid * ROWS_PER_SUB
    slot = [buf.at[pl.ds(u * CH, CH)] for u in range(NSLOT)]

    def rows(i):
        return pl.ds(row0 + i * CH, CH)

    def gstart(u, ci):
        pltpu.make_async_copy(x_hbm.at[rows(ci)], slot[u], gsem[u]).start()

    def gwait(u, ci):
        pltpu.make_async_copy(x_hbm.at[rows(ci)], slot[u], gsem[u]).wait()

    def sstart(u, ci):
        pltpu.make_async_copy(slot[u], out_hbm.at[rows(ci)], ssem[u]).start()

    def swait(u, ci):
        pltpu.make_async_copy(slot[u], out_hbm.at[rows(ci)], ssem[u]).wait()

    gstart(0, 0)
    gstart(1, 1)
    gwait(0, 0); sstart(0, 0); gstart(2, 2)
    gwait(1, 1); sstart(1, 1); gstart(3, 3)
    gwait(2, 2); sstart(2, 2); swait(0, 0); gstart(0, 4)
    gwait(3, 3); sstart(3, 3); swait(1, 1); gstart(1, 5)

    @pl.loop(1, NBLK - 1)
    def _pipe(i):
        base = NSLOT * i
        gwait(0, base + 0); sstart(0, base + 0); swait(2, base - 2); gstart(2, base + 2)
        gwait(1, base + 1); sstart(1, base + 1); swait(3, base - 1); gstart(3, base + 3)
        gwait(2, base + 2); sstart(2, base + 2); swait(0, base + 0); gstart(0, base + 4)
        gwait(3, base + 3); sstart(3, base + 3); swait(1, base + 1); gstart(1, base + 5)

    base = NCH - NSLOT
    gwait(0, base + 0); sstart(0, base + 0); swait(2, base - 2); gstart(2, base + 2)
    gwait(1, base + 1); sstart(1, base + 1); swait(3, base - 1); gstart(3, base + 3)
    gwait(2, base + 2); sstart(2, base + 2); swait(0, base + 0)
    gwait(3, base + 3); sstart(3, base + 3); swait(1, base + 1)
    swait(2, base + 2)
    swait(3, base + 3)

    plsc.subcore_barrier()

    sm = s % (NSUB // 2)
    @pl.when(sm < SCATTER_WORKERS)
    def _():
        b = 2 * c + s // (NSUB // 2)
        j = sm
        pltpu.sync_copy(seq_hbm.at[b], seq_v)
        pltpu.sync_copy(pf_hbm.at[pl.ds(j * PF_PER_SUB, PF_PER_SUB)], buf)
        lane = lax.iota(jnp.int32, 16)
        idx_v[...] = seq_v[...] + b * S + j * PF_PER_SUB + lane
        pltpu.async_copy(buf, out_hbm.at[idx_v], psem).wait()


def kernel(crossattn_emb, crossattn_seqlens, postfix_embeds):
    x2d = crossattn_emb.reshape(B * S, D)
    seq_bcast = jnp.broadcast_to(
        crossattn_seqlens.astype(jnp.int32)[:, None], (B, 16))
    out2d = _postfix_kernel(x2d, seq_bcast, postfix_embeds)
    return out2d.reshape(B, S, D)
